# baseline (speedup 1.0000x reference)
import jax
import jax.numpy as jnp
from jax.experimental import pallas as pl

_MESH_SIZE = 64
_NUM_ELEMS = _MESH_SIZE * _MESH_SIZE
_N_TASK = 50000
_D = 128
_HID = 128
_OUT = 2
_N_LAYERS_MID = 2

_EDGE_META = [
    ("task", "task", 200000),
    ("task", "task", 200000),
    ("task", "pe", 50000),
    ("pe", "task", 50000),
    ("router", "router", 65536),
    ("router", "pe", 16384),
    ("pe", "router", 16384),
]


def _graph_conv(x_src, x_dst, src, dst, W_rel, b_rel, W_root, n_dst):
    msgs = x_src[src]
    agg = jax.ops.segment_max(msgs, dst, num_segments=n_dst)
    cnt = jax.ops.segment_sum(jnp.ones((dst.shape[0],), dtype=jnp.float32), dst, num_segments=n_dst)
    agg = jnp.where(cnt[:, None] > 0, agg, 0.0)
    return agg @ W_rel + b_rel + x_dst @ W_root


def kernel(x_task, x_pe, x_router, ei_generates_for, ei_requires, ei_mapped_to, ei_rev_mapped_to, ei_link, ei_interface, ei_rev_interface, mask_task, pe_emb, router_emb, W_rel_mid, b_rel_mid, W_root_mid, W_rel_fin, b_rel_fin, W_root_fin):
    batch = x_pe.shape[0] // _NUM_ELEMS
    n_pe = x_pe.shape[0]
    sizes = {"task": _N_TASK, "pe": n_pe, "router": n_pe}
    x = {"task": x_task,
         "pe": jnp.tile(pe_emb, (batch, 1)),
         "router": jnp.tile(router_emb, (batch, 1))}
    eis = [ei_generates_for, ei_requires, ei_mapped_to, ei_rev_mapped_to, ei_link, ei_interface, ei_rev_interface]
    sorted_eis = []
    for ei in eis:
        order = jnp.argsort(ei[1])
        sorted_eis.append((ei[0][order], ei[1][order]))
    for l in range(_N_LAYERS_MID):
        out = {k: jnp.zeros((sizes[k], _HID), dtype=jnp.float32) for k in sizes}
        for i, (s, d, _) in enumerate(_EDGE_META):
            src, dst = sorted_eis[i]
            out[d] = out[d] + _graph_conv(x[s], x[d], src, dst, W_rel_mid[l, i], b_rel_mid[l, i], W_root_mid[l, i], sizes[d])
        x = {k: jax.nn.relu(v) for k, v in out.items()}
    out = {k: jnp.zeros((sizes[k], _OUT), dtype=jnp.float32) for k in sizes}
    for i, (s, d, _) in enumerate(_EDGE_META):
        src, dst = sorted_eis[i]
        out[d] = out[d] + _graph_conv(x[s], x[d], src, dst, W_rel_fin[i], b_rel_fin[i], W_root_fin[i], sizes[d])
    return (out["task"], out["pe"], out["router"])


# speedup vs baseline: 1.4256x; 1.4256x over previous
import functools

import jax
import jax.numpy as jnp
import numpy as np
from jax import lax
from jax.experimental import pallas as pl
from jax.experimental.pallas import tpu as pltpu
from jax.experimental.pallas import tpu_sc as plsc

_NUM_ELEMS = 64 * 64
_N_TASK = 50000
_D = 128
_OUTD = 2
_NMID = 2

_EMETA = [
    ("task", "task"),
    ("task", "task"),
    ("task", "pe"),
    ("pe", "task"),
    ("router", "router"),
    ("router", "pe"),
    ("pe", "router"),
]
_DST_RELS = {"task": [0, 1, 3], "pe": [2, 5], "router": [4, 6]}

_NW = 32
_RROWS = 768
_G = 128
_OCH = 128
_TBLF = 128


def _cdiv(a, b):
    return -(-a // b)


def _regpad(n):
    return _cdiv(n, 512) * 512


@functools.cache
def _make_sc_layer(key, esizes, mts, slot_offs, tot_rows, masks):
    mesh = plsc.VectorSubcoreMesh(core_axis_name="c", subcore_axis_name="s")
    mt_max = max(mts)
    nrel = len(esizes)

    @functools.partial(
        pl.kernel,
        mesh=mesh,
        out_type=jax.ShapeDtypeStruct((tot_rows * _D,), jnp.float32),
        scratch_types=[
            pltpu.VMEM((_RROWS * _D,), jnp.float32),
            pltpu.VMEM((_G, _D), jnp.float32),
            pltpu.VMEM((_G,), jnp.int32),
            pltpu.VMEM((_G,), jnp.int32),
            pltpu.VMEM((mt_max * _TBLF,), jnp.int32),
            pltpu.SemaphoreType.DMA,
        ],
    )
    def sc_layer(x_t, x_p, x_r, s0, d0, s1, d1, s2, d2, s3, d3, s4, d4,
                 s5, d5, s6, d6, tbl, out, acc, gbuf, idxv, dstv, tblv, gsem):
        wid = lax.axis_index("s") * 2 + lax.axis_index("c")
        neg = jnp.full((16,), -jnp.inf, dtype=jnp.float32)
        xs = {"task": x_t, "pe": x_p, "router": x_r}
        srcs = (s0, s1, s2, s3, s4, s5, s6)
        dsts = (d0, d1, d2, d3, d4, d5, d6)

        for r in range(nrel):
            x_h = xs[_EMETA[r][0]]
            src_h, dst_h = srcs[r], dsts[r]
            mt = mts[r]
            er = esizes[r]
            msk = masks[r]
            row0 = pl.multiple_of((slot_offs[r] + wid * mt) * _TBLF, 8)
            pltpu.sync_copy(tbl.at[pl.ds(row0, mt * _TBLF)],
                            tblv.at[pl.ds(0, mt * _TBLF)])

            def slot(j, _, x_h=x_h, src_h=src_h, dst_h=dst_h, er=er, msk=msk):
                trow = tblv[pl.ds(pl.multiple_of(j * _TBLF, 8), 16)]
                ea = trow[0]
                nch = trow[1]
                base = trow[2]
                orow = trow[3]
                nout = trow[4]

                def ms(q, _):
                    for v in range(_D // 16):
                        acc[pl.ds(q * _D + v * 16, 16)] = neg
                    return 0
                lax.fori_loop(0, _RROWS, ms, 0, unroll=4)

                def chunk(c, _):
                    e0 = pl.multiple_of(
                        jnp.minimum(ea + c * _G, er - _G), 8)
                    pltpu.sync_copy(src_h.at[pl.ds(e0, _G)], idxv)
                    pltpu.sync_copy(dst_h.at[pl.ds(e0, _G)], dstv)
                    if msk is not None:
                        for v in range(_G // 16):
                            sl = pl.ds(v * 16, 16)
                            idxv[sl] = idxv[sl] & msk
                    pltpu.async_copy(x_h.at[idxv], gbuf, gsem).wait()

                    def grp(k, _):
                        ld16 = dstv[pl.ds(k * 16, 16)] - base
                        for e2 in range(16):
                            ld = ld16[e2]

                            @pl.when((ld >= 0) & (ld < _RROWS))
                            def _(ld=ld, e2=e2, k=k):
                                e = k * 16 + e2
                                ro = ld * _D
                                for v in range(_D // 16):
                                    acc[pl.ds(ro + v * 16, 16)] = jnp.maximum(
                                        acc[pl.ds(ro + v * 16, 16)],
                                        gbuf[e, pl.ds(v * 16, 16)])
                        return 0
                    lax.fori_loop(0, _G // 16, grp, 0)
                    return 0
                lax.fori_loop(0, nch, chunk, 0)

                def oc(k, _):
                    pltpu.sync_copy(
                        acc.at[pl.ds(k * _OCH * _D, _OCH * _D)],
                        out.at[pl.ds(pl.multiple_of((orow + k * _OCH) * _D, 8),
                                     _OCH * _D)])
                    return 0
                lax.fori_loop(0, nout, oc, 0)
                return 0

            lax.fori_loop(0, mt, slot, 0)

    return sc_layer


def _tc_identity(x):
    def body(i_ref, o_ref):
        o_ref[...] = i_ref[...]
    return pl.pallas_call(
        body, out_shape=jax.ShapeDtypeStruct(x.shape, x.dtype))(x)


def _tc_lin(agg, agg_offs, x, w_stack, b8, n, relu):
    nrel = len(agg_offs)
    nb = _cdiv(n, 512)

    def body(*refs):
        a_refs = refs[:nrel]
        xr, wr, br, outr = refs[nrel:]
        acc = jnp.dot(xr[...], wr[nrel], preferred_element_type=jnp.float32)
        for r in range(nrel):
            a = a_refs[r][...]
            a = jnp.where(a > -jnp.inf, a, 0.0)
            acc = acc + jnp.dot(a, wr[r], preferred_element_type=jnp.float32)
        acc = acc + br[0:1, :]
        if relu:
            acc = jnp.maximum(acc, 0.0)
        outr[...] = acc

    in_specs = [
        pl.BlockSpec((512, _D), lambda i, o=off // 512: (i + o, 0))
        for off in agg_offs
    ] + [
        pl.BlockSpec((512, _D), lambda i: (i, 0)),
        pl.BlockSpec((nrel + 1, _D, _D), lambda i: (0, 0, 0)),
        pl.BlockSpec((8, _D), lambda i: (0, 0)),
    ]
    return pl.pallas_call(
        body,
        grid=(nb,),
        in_specs=in_specs,
        out_specs=pl.BlockSpec((512, _D), lambda i: (i, 0)),
        out_shape=jax.ShapeDtypeStruct((n, _D), jnp.float32),
    )(*([agg] * nrel + [x, w_stack, b8]))


def kernel(x_task, x_pe, x_router, ei_generates_for, ei_requires, ei_mapped_to,
           ei_rev_mapped_to, ei_link, ei_interface, ei_rev_interface,
           mask_task, pe_emb, router_emb, W_rel_mid, b_rel_mid, W_root_mid,
           W_rel_fin, b_rel_fin, W_root_fin):
    n_pe = x_pe.shape[0]
    sizes = {"task": _N_TASK, "pe": n_pe, "router": n_pe}
    eis = [ei_generates_for, ei_requires, ei_mapped_to, ei_rev_mapped_to,
           ei_link, ei_interface, ei_rev_interface]
    nrel = len(_EMETA)
    esizes = tuple(int(e.shape[1]) for e in eis)
    nranges = [_cdiv(sizes[d], _RROWS) for (_, d) in _EMETA]
    regpad = [_regpad(sizes[d]) for (_, d) in _EMETA]
    reg_off = [0]
    for rp in regpad:
        reg_off.append(reg_off[-1] + rp)
    tot_rows = reg_off[-1]

    mts = tuple(max(1, _cdiv(nr, _NW)) for nr in nranges)
    slot_offs, so = [], 0
    for r in range(nrel):
        slot_offs.append(so)
        so += _NW * mts[r]
    tot_slots = so

    srcs_s, dsts_s = [], []
    for r in range(nrel):
        dst_s, src_s = lax.sort_key_val(eis[r][1], eis[r][0])
        srcs_s.append(src_s)
        dsts_s.append(dst_s)

    base_np = np.zeros((tot_slots,), np.int32)
    orow_np = np.zeros((tot_slots,), np.int32)
    nout_np = np.zeros((tot_slots,), np.int32)
    valid_np = np.zeros((tot_slots,), np.int32)
    sel_rows = np.zeros((tot_slots,), np.int64)
    for r in range(nrel):
        for k in range(nranges[r]):
            w, j = k % _NW, k // _NW
            s = slot_offs[r] + w * mts[r] + j
            base = k * _RROWS
            base_np[s] = base
            orow_np[s] = reg_off[r] + base
            nout_np[s] = min(_RROWS, regpad[r] - base) // _OCH
            valid_np[s] = 1
            sel_rows[s] = k

    tbl_rows = []
    for r in range(nrel):
        bounds = (jnp.arange(nranges[r] + 1, dtype=jnp.int32) * _RROWS)
        ss = jnp.searchsorted(dsts_s[r], bounds).astype(jnp.int32)
        ea = (ss[:-1] // 8) * 8
        nch = (ss[1:] - ea + (_G - 1)) // _G
        s0, s1 = slot_offs[r], slot_offs[r] + _NW * mts[r]
        sel = sel_rows[s0:s1]
        v = jnp.asarray(valid_np[s0:s1])
        row = jnp.stack([ea[sel] * v, nch[sel] * v,
                         jnp.asarray(base_np[s0:s1]),
                         jnp.asarray(orow_np[s0:s1]),
                         jnp.asarray(nout_np[s0:s1]) * v,
                         v], axis=1)
        tbl_rows.append(row)
    tbl16 = jnp.concatenate(tbl_rows, axis=0).astype(jnp.int32)
    tbl = jnp.zeros((tot_slots, _TBLF), jnp.int32)
    tbl = tbl.at[:, :6].set(tbl16)
    tbl = _tc_identity(tbl).reshape(-1)

    sc0 = _make_sc_layer("L0", esizes, mts, tuple(slot_offs), tot_rows,
                         (None, None, None, _NUM_ELEMS - 1, _NUM_ELEMS - 1,
                          _NUM_ELEMS - 1, _NUM_ELEMS - 1))
    sc12 = _make_sc_layer("L12", esizes, mts, tuple(slot_offs), tot_rows,
                          (None, None, None, None, None, None, None))

    def sc_call(which, x_t, x_p, x_r):
        args = [x_t, x_p, x_r]
        for r in range(nrel):
            args += [srcs_s[r], dsts_s[r]]
        args.append(tbl)
        return which(*args).reshape(tot_rows, _D)

    def mid_w(l, d):
        rels = _DST_RELS[d]
        w = jnp.stack([W_rel_mid[l, i] for i in rels]
                      + [sum(W_root_mid[l, i] for i in rels)])
        b = sum(b_rel_mid[l, i] for i in rels)
        return w, jnp.broadcast_to(b, (8, _D))

    def fin_w(d):
        rels = _DST_RELS[d]
        w = jnp.stack([jnp.pad(W_rel_fin[i], ((0, 0), (0, _D - _OUTD)))
                       for i in rels]
                      + [jnp.pad(sum(W_root_fin[i] for i in rels),
                                 ((0, 0), (0, _D - _OUTD)))])
        b = jnp.pad(sum(b_rel_fin[i] for i in rels), (0, _D - _OUTD))
        return w, jnp.broadcast_to(b, (8, _D))

    x = {"task": x_task, "pe": None, "router": None}
    for l in range(_NMID):
        if l == 0:
            agg = sc_call(sc0, x_task, pe_emb, router_emb)
            xroot = {"task": x_task,
                     "pe": jnp.tile(pe_emb, (n_pe // _NUM_ELEMS, 1)),
                     "router": jnp.tile(router_emb, (n_pe // _NUM_ELEMS, 1))}
        else:
            agg = sc_call(sc12, x["task"], x["pe"], x["router"])
            xroot = x
        nx = {}
        for d in ("task", "pe", "router"):
            w, b8 = mid_w(l, d)
            offs = [reg_off[r] for r in _DST_RELS[d]]
            nx[d] = _tc_lin(agg, offs, xroot[d], w, b8, sizes[d], relu=True)
        x = nx
    agg = sc_call(sc12, x["task"], x["pe"], x["router"])
    outs = []
    for d in ("task", "pe", "router"):
        w, b8 = fin_w(d)
        offs = [reg_off[r] for r in _DST_RELS[d]]
        o = _tc_lin(agg, offs, x[d], w, b8, sizes[d], relu=False)
        outs.append(o[:, :_OUTD])
    return tuple(outs)


# speedup vs baseline: 1.5177x; 1.0647x over previous
import functools

import jax
import jax.numpy as jnp
import numpy as np
from jax import lax
from jax.experimental import pallas as pl
from jax.experimental.pallas import tpu as pltpu
from jax.experimental.pallas import tpu_sc as plsc

_NUM_ELEMS = 64 * 64
_N_TASK = 50000
_D = 128
_OUTD = 2
_NMID = 2

_EMETA = [
    ("task", "task"),
    ("task", "task"),
    ("task", "pe"),
    ("pe", "task"),
    ("router", "router"),
    ("router", "pe"),
    ("pe", "router"),
]
_DST_RELS = {"task": [0, 1, 3], "pe": [2, 5], "router": [4, 6]}

_NW = 32
_RROWS = 768
_G = 64
_OCH = 128
_TBLF = 128


def _cdiv(a, b):
    return -(-a // b)


def _regpad(n):
    return _cdiv(n, 512) * 512


@functools.cache
def _make_sc_layer(key, esizes, mts, slot_offs, tot_rows, masks):
    mesh = plsc.VectorSubcoreMesh(core_axis_name="c", subcore_axis_name="s")
    mt_max = max(mts)
    nrel = len(esizes)

    @functools.partial(
        pl.kernel,
        mesh=mesh,
        out_type=jax.ShapeDtypeStruct((tot_rows * _D,), jnp.float32),
        scratch_types=[
            pltpu.VMEM((_RROWS * _D,), jnp.float32),
            pltpu.VMEM((_G, _D), jnp.float32),
            pltpu.VMEM((_G, _D), jnp.float32),
            pltpu.VMEM((_G,), jnp.int32),
            pltpu.VMEM((_G,), jnp.int32),
            pltpu.VMEM((_G,), jnp.int32),
            pltpu.VMEM((_G,), jnp.int32),
            pltpu.VMEM((mt_max * _TBLF,), jnp.int32),
            pltpu.SemaphoreType.DMA,
            pltpu.SemaphoreType.DMA,
        ],
    )
    def sc_layer(x_t, x_p, x_r, s0, d0, s1, d1, s2, d2, s3, d3, s4, d4,
                 s5, d5, s6, d6, tbl, out, acc, gbuf0, gbuf1, idxv0, idxv1,
                 dstv0, dstv1, tblv, gsem0, gsem1):
        wid = lax.axis_index("s") * 2 + lax.axis_index("c")
        neg = jnp.full((16,), -jnp.inf, dtype=jnp.float32)
        xs = {"task": x_t, "pe": x_p, "router": x_r}
        srcs = (s0, s1, s2, s3, s4, s5, s6)
        dsts = (d0, d1, d2, d3, d4, d5, d6)

        for r in range(nrel):
            x_h = xs[_EMETA[r][0]]
            src_h, dst_h = srcs[r], dsts[r]
            mt = mts[r]
            er = esizes[r]
            msk = masks[r]
            row0 = pl.multiple_of((slot_offs[r] + wid * mt) * _TBLF, 8)
            pltpu.sync_copy(tbl.at[pl.ds(row0, mt * _TBLF)],
                            tblv.at[pl.ds(0, mt * _TBLF)])

            def slot(j, _, x_h=x_h, src_h=src_h, dst_h=dst_h, er=er, msk=msk):
                trow = tblv[pl.ds(pl.multiple_of(j * _TBLF, 8), 16)]
                ea = trow[0]
                nch = trow[1]
                base = trow[2]
                orow = trow[3]
                nout = trow[4]

                def ms(q, _):
                    for v in range(_D // 16):
                        acc[pl.ds(q * _D + v * 16, 16)] = neg
                    return 0
                lax.fori_loop(0, _RROWS, ms, 0, unroll=4)

                bufs = ((gbuf0, idxv0, dstv0, gsem0),
                        (gbuf1, idxv1, dstv1, gsem1))

                def gate(cond, body):
                    lax.fori_loop(0, cond.astype(jnp.int32),
                                  lambda _i, _c: (body(), 0)[1], 0)

                def start(c, b, x_h=x_h, src_h=src_h, dst_h=dst_h, er=er,
                          msk=msk):
                    gbuf, idxv, dstv, gsem = bufs[b]
                    e0 = pl.multiple_of(
                        jnp.minimum(ea + c * _G, er - _G), 8)
                    pltpu.sync_copy(src_h.at[pl.ds(e0, _G)], idxv)
                    pltpu.sync_copy(dst_h.at[pl.ds(e0, _G)], dstv)
                    if msk is not None:
                        for v in range(_G // 16):
                            sl = pl.ds(v * 16, 16)
                            idxv[sl] = idxv[sl] & msk
                    pltpu.make_async_copy(x_h.at[idxv], gbuf, gsem).start()

                def process(b):
                    gbuf, idxv, dstv, gsem = bufs[b]
                    pltpu.make_async_copy(x_h.at[idxv], gbuf, gsem).wait()

                    def grp(k, _):
                        ld16 = dstv[pl.ds(k * 16, 16)] - base
                        for e2 in range(16):
                            ld = ld16[e2]

                            @pl.when((ld >= 0) & (ld < _RROWS))
                            def _(ld=ld, e2=e2, k=k):
                                e = k * 16 + e2
                                ro = ld * _D
                                for v in range(_D // 16):
                                    acc[pl.ds(ro + v * 16, 16)] = jnp.maximum(
                                        acc[pl.ds(ro + v * 16, 16)],
                                        gbuf[e, pl.ds(v * 16, 16)])
                        return 0
                    lax.fori_loop(0, _G // 16, grp, 0)

                gate(nch > 0, lambda: start(jnp.int32(0), 0))
                gate(nch > 1, lambda: start(jnp.int32(1), 1))

                def pair(p, _):
                    for b in range(2):
                        c = p * 2 + b
                        gate(c < nch, functools.partial(process, b))
                        gate(c + 2 < nch,
                             lambda c=c, b=b: start(c + 2, b))
                    return 0
                lax.fori_loop(0, (nch + 1) // 2, pair, 0)

                def oc(k, _):
                    pltpu.sync_copy(
                        acc.at[pl.ds(k * _OCH * _D, _OCH * _D)],
                        out.at[pl.ds(pl.multiple_of((orow + k * _OCH) * _D, 8),
                                     _OCH * _D)])
                    return 0
                lax.fori_loop(0, nout, oc, 0)
                return 0

            lax.fori_loop(0, mt, slot, 0)

    return sc_layer


def _tc_identity(x):
    def body(i_ref, o_ref):
        o_ref[...] = i_ref[...]
    return pl.pallas_call(
        body, out_shape=jax.ShapeDtypeStruct(x.shape, x.dtype))(x)


def _tc_lin(agg, agg_offs, x, w_stack, b8, n, relu):
    nrel = len(agg_offs)
    nb = _cdiv(n, 512)

    def body(*refs):
        a_refs = refs[:nrel]
        xr, wr, br, outr = refs[nrel:]
        acc = jnp.dot(xr[...], wr[nrel], preferred_element_type=jnp.float32)
        for r in range(nrel):
            a = a_refs[r][...]
            a = jnp.where(a > -jnp.inf, a, 0.0)
            acc = acc + jnp.dot(a, wr[r], preferred_element_type=jnp.float32)
        acc = acc + br[0:1, :]
        if relu:
            acc = jnp.maximum(acc, 0.0)
        outr[...] = acc

    in_specs = [
        pl.BlockSpec((512, _D), lambda i, o=off // 512: (i + o, 0))
        for off in agg_offs
    ] + [
        pl.BlockSpec((512, _D), lambda i: (i, 0)),
        pl.BlockSpec((nrel + 1, _D, _D), lambda i: (0, 0, 0)),
        pl.BlockSpec((8, _D), lambda i: (0, 0)),
    ]
    return pl.pallas_call(
        body,
        grid=(nb,),
        in_specs=in_specs,
        out_specs=pl.BlockSpec((512, _D), lambda i: (i, 0)),
        out_shape=jax.ShapeDtypeStruct((n, _D), jnp.float32),
    )(*([agg] * nrel + [x, w_stack, b8]))


def kernel(x_task, x_pe, x_router, ei_generates_for, ei_requires, ei_mapped_to,
           ei_rev_mapped_to, ei_link, ei_interface, ei_rev_interface,
           mask_task, pe_emb, router_emb, W_rel_mid, b_rel_mid, W_root_mid,
           W_rel_fin, b_rel_fin, W_root_fin):
    n_pe = x_pe.shape[0]
    sizes = {"task": _N_TASK, "pe": n_pe, "router": n_pe}
    eis = [ei_generates_for, ei_requires, ei_mapped_to, ei_rev_mapped_to,
           ei_link, ei_interface, ei_rev_interface]
    nrel = len(_EMETA)
    esizes = tuple(int(e.shape[1]) for e in eis)
    nranges = [_cdiv(sizes[d], _RROWS) for (_, d) in _EMETA]
    regpad = [_regpad(sizes[d]) for (_, d) in _EMETA]
    reg_off = [0]
    for rp in regpad:
        reg_off.append(reg_off[-1] + rp)
    tot_rows = reg_off[-1]

    mts = tuple(max(1, _cdiv(nr, _NW)) for nr in nranges)
    slot_offs, so = [], 0
    for r in range(nrel):
        slot_offs.append(so)
        so += _NW * mts[r]
    tot_slots = so

    srcs_s, dsts_s = [], []
    for r in range(nrel):
        dst_s, src_s = lax.sort_key_val(eis[r][1], eis[r][0])
        srcs_s.append(src_s)
        dsts_s.append(dst_s)

    base_np = np.zeros((tot_slots,), np.int32)
    orow_np = np.zeros((tot_slots,), np.int32)
    nout_np = np.zeros((tot_slots,), np.int32)
    valid_np = np.zeros((tot_slots,), np.int32)
    sel_rows = np.zeros((tot_slots,), np.int64)
    for r in range(nrel):
        for k in range(nranges[r]):
            w, j = k % _NW, k // _NW
            s = slot_offs[r] + w * mts[r] + j
            base = k * _RROWS
            base_np[s] = base
            orow_np[s] = reg_off[r] + base
            nout_np[s] = min(_RROWS, regpad[r] - base) // _OCH
            valid_np[s] = 1
            sel_rows[s] = k

    tbl_rows = []
    for r in range(nrel):
        bounds = (jnp.arange(nranges[r] + 1, dtype=jnp.int32) * _RROWS)
        ss = jnp.searchsorted(dsts_s[r], bounds).astype(jnp.int32)
        ea = (ss[:-1] // 8) * 8
        nch = (ss[1:] - ea + (_G - 1)) // _G
        s0, s1 = slot_offs[r], slot_offs[r] + _NW * mts[r]
        sel = sel_rows[s0:s1]
        v = jnp.asarray(valid_np[s0:s1])
        row = jnp.stack([ea[sel] * v, nch[sel] * v,
                         jnp.asarray(base_np[s0:s1]),
                         jnp.asarray(orow_np[s0:s1]),
                         jnp.asarray(nout_np[s0:s1]) * v,
                         v], axis=1)
        tbl_rows.append(row)
    tbl16 = jnp.concatenate(tbl_rows, axis=0).astype(jnp.int32)
    tbl = jnp.zeros((tot_slots, _TBLF), jnp.int32)
    tbl = tbl.at[:, :6].set(tbl16)
    tbl = _tc_identity(tbl).reshape(-1)

    sc0 = _make_sc_layer("L0", esizes, mts, tuple(slot_offs), tot_rows,
                         (None, None, None, _NUM_ELEMS - 1, _NUM_ELEMS - 1,
                          _NUM_ELEMS - 1, _NUM_ELEMS - 1))
    sc12 = _make_sc_layer("L12", esizes, mts, tuple(slot_offs), tot_rows,
                          (None, None, None, None, None, None, None))

    def sc_call(which, x_t, x_p, x_r):
        args = [x_t, x_p, x_r]
        for r in range(nrel):
            args += [srcs_s[r], dsts_s[r]]
        args.append(tbl)
        return which(*args).reshape(tot_rows, _D)

    def mid_w(l, d):
        rels = _DST_RELS[d]
        w = jnp.stack([W_rel_mid[l, i] for i in rels]
                      + [sum(W_root_mid[l, i] for i in rels)])
        b = sum(b_rel_mid[l, i] for i in rels)
        return w, jnp.broadcast_to(b, (8, _D))

    def fin_w(d):
        rels = _DST_RELS[d]
        w = jnp.stack([jnp.pad(W_rel_fin[i], ((0, 0), (0, _D - _OUTD)))
                       for i in rels]
                      + [jnp.pad(sum(W_root_fin[i] for i in rels),
                                 ((0, 0), (0, _D - _OUTD)))])
        b = jnp.pad(sum(b_rel_fin[i] for i in rels), (0, _D - _OUTD))
        return w, jnp.broadcast_to(b, (8, _D))

    x = {"task": x_task, "pe": None, "router": None}
    for l in range(_NMID):
        if l == 0:
            agg = sc_call(sc0, x_task, pe_emb, router_emb)
            xroot = {"task": x_task,
                     "pe": jnp.tile(pe_emb, (n_pe // _NUM_ELEMS, 1)),
                     "router": jnp.tile(router_emb, (n_pe // _NUM_ELEMS, 1))}
        else:
            agg = sc_call(sc12, x["task"], x["pe"], x["router"])
            xroot = x
        nx = {}
        for d in ("task", "pe", "router"):
            w, b8 = mid_w(l, d)
            offs = [reg_off[r] for r in _DST_RELS[d]]
            nx[d] = _tc_lin(agg, offs, xroot[d], w, b8, sizes[d], relu=True)
        x = nx
    agg = sc_call(sc12, x["task"], x["pe"], x["router"])
    outs = []
    for d in ("task", "pe", "router"):
        w, b8 = fin_w(d)
        offs = [reg_off[r] for r in _DST_RELS[d]]
        o = _tc_lin(agg, offs, x[d], w, b8, sizes[d], relu=False)
        outs.append(o[:, :_OUTD])
    return tuple(outs)
